# Initial kernel scaffold; baseline (speedup 1.0000x reference)
import jax
import jax.numpy as jnp
from jax.experimental import pallas as pl


def kernel(img, aug_img, Wp, W1, W2, Ws, W1e, W2e, Wse, centroid, queue):
    raise NotImplementedError("write your pallas kernel here")


# speedup vs baseline: 1.9954x; 1.9954x over previous
import functools

import jax
import jax.numpy as jnp
from jax.experimental import pallas as pl

B, C, HW, P = 16, 3, 384, 16
HP = HW // P
T = HP * HP
FEAT, HID = 384, 64
K, NS = 512, 100
MOM, TS = 0.99, 0.07

N_TOK = B * T
RA = 512
GA = N_TOK // RA
NQ = K * NS
RB = 2048
GB = NQ // RB

_NEG_BIG = -3.0e38


def _norm_rows(x):
    n = jnp.sqrt(jnp.sum(x * x, axis=-1, keepdims=True))
    return x / jnp.clip(n, 1e-12)


def _kernel_a(tok1_ref, tok2_ref, wp_ref, w1_ref, w2_ref, ws_ref,
              w1e_ref, w2e_ref, wse_ref, cent_ref,
              nz1_ref, z1_ref, z2_ref, idx_ref, gap_ref, mse_ref):
    i = pl.program_id(0)
    f32 = jnp.float32

    x1 = jnp.dot(tok1_ref[...], wp_ref[...], preferred_element_type=f32)
    h1 = jnp.dot(jnp.maximum(jnp.dot(x1, w1_ref[...], preferred_element_type=f32), 0.0),
                 w2_ref[...], preferred_element_type=f32)
    h1 = h1 + jnp.dot(x1, ws_ref[...], preferred_element_type=f32)
    z1_ref[...] = h1
    nz1 = _norm_rows(h1)
    nz1_ref[...] = nz1

    w1n = MOM * w1e_ref[...] + (1.0 - MOM) * w1_ref[...]
    w2n = MOM * w2e_ref[...] + (1.0 - MOM) * w2_ref[...]
    wsn = MOM * wse_ref[...] + (1.0 - MOM) * ws_ref[...]
    x2 = jnp.dot(tok2_ref[...], wp_ref[...], preferred_element_type=f32)
    h2 = jnp.dot(jnp.maximum(jnp.dot(x2, w1n, preferred_element_type=f32), 0.0),
                 w2n, preferred_element_type=f32)
    h2 = h2 + jnp.dot(x2, wsn, preferred_element_type=f32)
    z2_ref[...] = h2
    nz2 = _norm_rows(h2)

    d = nz1 - nz2
    mse_part = jnp.sum(d * d) * (1.0 / (N_TOK * HID))

    cn = _norm_rows(cent_ref[...])
    cn2 = jnp.sum(cn * cn, axis=1)
    rn2 = jnp.sum(nz1 * nz1, axis=1, keepdims=True)
    s = jax.lax.dot_general(nz1, cn, (((1,), (1,)), ((), ())),
                            preferred_element_type=f32)
    neg = 2.0 * s - rn2 - cn2[None, :]
    m1 = jnp.max(neg, axis=1, keepdims=True)
    col = jax.lax.broadcasted_iota(jnp.int32, (RA, K), 1)
    idxv = jnp.min(jnp.where(neg == m1, col, K), axis=1)
    neg2 = jnp.where(col == idxv[:, None], _NEG_BIG, neg)
    m2 = jnp.max(neg2, axis=1)
    idx_ref[0, 0, :] = idxv
    gap_ref[0, 0, :] = m1[:, 0] - m2

    @pl.when(i == 0)
    def _():
        mse_ref[...] = mse_part.reshape(1, 1)

    @pl.when(i > 0)
    def _():
        mse_ref[...] += mse_part.reshape(1, 1)


def _kernel_b(q_ref, cent_ref, nce_ref):
    i = pl.program_id(0)
    f32 = jnp.float32

    qn = _norm_rows(q_ref[...])
    cn = _norm_rows(cent_ref[...])
    logits = jax.lax.dot_general(qn, cn, (((1,), (1,)), ((), ())),
                                 preferred_element_type=f32) * (1.0 / TS)
    m = jnp.max(logits, axis=1, keepdims=True)
    lse = jnp.log(jnp.sum(jnp.exp(logits - m), axis=1)) + m[:, 0]

    rows = i * RB + jax.lax.broadcasted_iota(jnp.int32, (RB, 1), 0)
    col = jax.lax.broadcasted_iota(jnp.int32, (RB, K), 1)
    hit = (rows >= NS * col) & (rows < NS * (col + 1))
    lab_logit = jnp.sum(jnp.where(hit, logits, 0.0), axis=1)
    part = jnp.sum(lse - lab_logit) * (1.0 / NQ)

    @pl.when(i == 0)
    def _():
        nce_ref[...] = part.reshape(1, 1)

    @pl.when(i > 0)
    def _():
        nce_ref[...] += part.reshape(1, 1)


@functools.partial(jax.jit)
def kernel(img, aug_img, Wp, W1, W2, Ws, W1e, W2e, Wse, centroid, queue):
    def _tok(x):
        x = x.reshape(B, C, HP, P, HP, P).transpose(0, 2, 4, 1, 3, 5)
        return x.reshape(N_TOK, C * P * P)

    tok1 = _tok(img)
    tok2 = _tok(aug_img)

    full = lambda shp: pl.BlockSpec(shp, lambda i: (0,) * len(shp))
    rowblk = pl.BlockSpec((RA, HID), lambda i: (i, 0))

    nz1, z1, z2, idx3, gap3, mse = pl.pallas_call(
        _kernel_a,
        grid=(GA,),
        in_specs=[
            pl.BlockSpec((RA, C * P * P), lambda i: (i, 0)),
            pl.BlockSpec((RA, C * P * P), lambda i: (i, 0)),
            full((C * P * P, FEAT)),
            full((FEAT, FEAT)), full((FEAT, HID)), full((FEAT, HID)),
            full((FEAT, FEAT)), full((FEAT, HID)), full((FEAT, HID)),
            full((K, HID)),
        ],
        out_specs=[
            rowblk, rowblk, rowblk,
            pl.BlockSpec((1, 1, RA), lambda i: (i, 0, 0)),
            pl.BlockSpec((1, 1, RA), lambda i: (i, 0, 0)),
            pl.BlockSpec((1, 1), lambda i: (0, 0)),
        ],
        out_shape=[
            jax.ShapeDtypeStruct((N_TOK, HID), jnp.float32),
            jax.ShapeDtypeStruct((N_TOK, HID), jnp.float32),
            jax.ShapeDtypeStruct((N_TOK, HID), jnp.float32),
            jax.ShapeDtypeStruct((GA, 1, RA), jnp.int32),
            jax.ShapeDtypeStruct((GA, 1, RA), jnp.float32),
            jax.ShapeDtypeStruct((1, 1), jnp.float32),
        ],
    )(tok1, tok2, Wp, W1, W2, Ws, W1e, W2e, Wse, centroid)

    qflat = queue.reshape(NQ, HID)
    nce = pl.pallas_call(
        _kernel_b,
        grid=(GB,),
        in_specs=[
            pl.BlockSpec((RB, HID), lambda i: (i, 0)),
            full((K, HID)),
        ],
        out_specs=pl.BlockSpec((1, 1), lambda i: (0, 0)),
        out_shape=jax.ShapeDtypeStruct((1, 1), jnp.float32),
    )(qflat, centroid)

    out = nz1.reshape(B, HP, HP, HID).transpose(0, 3, 1, 2)
    z1o = z1.reshape(B, HP, HP, HID).transpose(0, 3, 1, 2)
    z2o = z2.reshape(B, HP, HP, HID).transpose(0, 3, 1, 2)
    return (out, z1o, z2o, mse[0, 0], nce[0, 0],
            idx3.reshape(N_TOK), gap3.reshape(N_TOK))


# speedup vs baseline: 3.2505x; 1.6290x over previous
import functools

import jax
import jax.numpy as jnp
from jax.experimental import pallas as pl

B, C, HW, P = 16, 3, 384, 16
HP = HW // P
T = HP * HP
FEAT, HID = 384, 64
K, NS = 512, 100
MOM, TS = 0.99, 0.07

N_TOK = B * T
RA = T
GA = B
NQ = K * NS
RB = 2048
GB = NQ // RB

_NEG_BIG = -3.0e38


def _norm_rows(x):
    n = jnp.sqrt(jnp.sum(x * x, axis=-1, keepdims=True))
    return x / jnp.clip(n, 1e-12)


def _patch_tok(I):
    return I.reshape(C, HP, P, HP, P).transpose(1, 3, 0, 2, 4).reshape(T, C * P * P)


def _kernel_a(img_ref, aug_ref, wp_ref, w1_ref, w2_ref, ws_ref,
              w1e_ref, w2e_ref, wse_ref, cent_ref,
              nz1_ref, z1_ref, z2_ref, idx_ref, gap_ref, mse_ref):
    i = pl.program_id(0)
    f32 = jnp.float32

    x1 = jnp.dot(_patch_tok(img_ref[0]), wp_ref[...], preferred_element_type=f32)
    h1 = jnp.dot(jnp.maximum(jnp.dot(x1, w1_ref[...], preferred_element_type=f32), 0.0),
                 w2_ref[...], preferred_element_type=f32)
    h1 = h1 + jnp.dot(x1, ws_ref[...], preferred_element_type=f32)
    z1_ref[...] = h1
    nz1 = _norm_rows(h1)
    nz1_ref[...] = nz1

    w1n = MOM * w1e_ref[...] + (1.0 - MOM) * w1_ref[...]
    w2n = MOM * w2e_ref[...] + (1.0 - MOM) * w2_ref[...]
    wsn = MOM * wse_ref[...] + (1.0 - MOM) * ws_ref[...]
    x2 = jnp.dot(_patch_tok(aug_ref[0]), wp_ref[...], preferred_element_type=f32)
    h2 = jnp.dot(jnp.maximum(jnp.dot(x2, w1n, preferred_element_type=f32), 0.0),
                 w2n, preferred_element_type=f32)
    h2 = h2 + jnp.dot(x2, wsn, preferred_element_type=f32)
    z2_ref[...] = h2
    nz2 = _norm_rows(h2)

    d = nz1 - nz2
    mse_part = jnp.sum(d * d) * (1.0 / (N_TOK * HID))

    cn = _norm_rows(cent_ref[...])
    cn2 = jnp.sum(cn * cn, axis=1)
    rn2 = jnp.sum(nz1 * nz1, axis=1, keepdims=True)
    s = jax.lax.dot_general(nz1, cn, (((1,), (1,)), ((), ())),
                            preferred_element_type=f32)
    neg = 2.0 * s - rn2 - cn2[None, :]
    m1 = jnp.max(neg, axis=1, keepdims=True)
    col = jax.lax.broadcasted_iota(jnp.int32, (RA, K), 1)
    idxv = jnp.min(jnp.where(neg == m1, col, K), axis=1)
    neg2 = jnp.where(col == idxv[:, None], _NEG_BIG, neg)
    m2 = jnp.max(neg2, axis=1)
    idx_ref[0, 0, :] = idxv
    gap_ref[0, 0, :] = m1[:, 0] - m2

    @pl.when(i == 0)
    def _():
        mse_ref[...] = mse_part.reshape(1, 1)

    @pl.when(i > 0)
    def _():
        mse_ref[...] += mse_part.reshape(1, 1)


def _kernel_b(q_ref, cent_ref, nce_ref):
    i = pl.program_id(0)
    f32 = jnp.float32

    qn = _norm_rows(q_ref[...])
    cn = _norm_rows(cent_ref[...])
    logits = jax.lax.dot_general(qn, cn, (((1,), (1,)), ((), ())),
                                 preferred_element_type=f32) * (1.0 / TS)
    m = jnp.max(logits, axis=1, keepdims=True)
    lse = jnp.log(jnp.sum(jnp.exp(logits - m), axis=1)) + m[:, 0]

    rows = i * RB + jax.lax.broadcasted_iota(jnp.int32, (RB, 1), 0)
    col = jax.lax.broadcasted_iota(jnp.int32, (RB, K), 1)
    hit = (rows >= NS * col) & (rows < NS * (col + 1))
    lab_logit = jnp.sum(jnp.where(hit, logits, 0.0), axis=1)
    part = jnp.sum(lse - lab_logit) * (1.0 / NQ)

    @pl.when(i == 0)
    def _():
        nce_ref[...] = part.reshape(1, 1)

    @pl.when(i > 0)
    def _():
        nce_ref[...] += part.reshape(1, 1)


@functools.partial(jax.jit)
def kernel(img, aug_img, Wp, W1, W2, Ws, W1e, W2e, Wse, centroid, queue):
    full = lambda shp: pl.BlockSpec(shp, lambda i: (0,) * len(shp))
    rowblk = pl.BlockSpec((RA, HID), lambda i: (i, 0))

    nz1, z1, z2, idx3, gap3, mse = pl.pallas_call(
        _kernel_a,
        grid=(GA,),
        in_specs=[
            pl.BlockSpec((1, C, HW, HW), lambda i: (i, 0, 0, 0)),
            pl.BlockSpec((1, C, HW, HW), lambda i: (i, 0, 0, 0)),
            full((C * P * P, FEAT)),
            full((FEAT, FEAT)), full((FEAT, HID)), full((FEAT, HID)),
            full((FEAT, FEAT)), full((FEAT, HID)), full((FEAT, HID)),
            full((K, HID)),
        ],
        out_specs=[
            rowblk, rowblk, rowblk,
            pl.BlockSpec((1, 1, RA), lambda i: (i, 0, 0)),
            pl.BlockSpec((1, 1, RA), lambda i: (i, 0, 0)),
            pl.BlockSpec((1, 1), lambda i: (0, 0)),
        ],
        out_shape=[
            jax.ShapeDtypeStruct((N_TOK, HID), jnp.float32),
            jax.ShapeDtypeStruct((N_TOK, HID), jnp.float32),
            jax.ShapeDtypeStruct((N_TOK, HID), jnp.float32),
            jax.ShapeDtypeStruct((GA, 1, RA), jnp.int32),
            jax.ShapeDtypeStruct((GA, 1, RA), jnp.float32),
            jax.ShapeDtypeStruct((1, 1), jnp.float32),
        ],
    )(img, aug_img, Wp, W1, W2, Ws, W1e, W2e, Wse, centroid)

    qflat = queue.reshape(NQ, HID)
    nce = pl.pallas_call(
        _kernel_b,
        grid=(GB,),
        in_specs=[
            pl.BlockSpec((RB, HID), lambda i: (i, 0)),
            full((K, HID)),
        ],
        out_specs=pl.BlockSpec((1, 1), lambda i: (0, 0)),
        out_shape=jax.ShapeDtypeStruct((1, 1), jnp.float32),
    )(qflat, centroid)

    out = nz1.reshape(B, HP, HP, HID).transpose(0, 3, 1, 2)
    z1o = z1.reshape(B, HP, HP, HID).transpose(0, 3, 1, 2)
    z2o = z2.reshape(B, HP, HP, HID).transpose(0, 3, 1, 2)
    return (out, z1o, z2o, mse[0, 0], nce[0, 0],
            idx3.reshape(N_TOK), gap3.reshape(N_TOK))
